# Initial kernel scaffold; baseline (speedup 1.0000x reference)
import jax
import jax.numpy as jnp
from jax.experimental import pallas as pl


def kernel(frequencies, star_indices, delta_nu_hard, delta_nu_corr):
    raise NotImplementedError("write your pallas kernel here")


# speedup vs baseline: 150.3801x; 150.3801x over previous
import functools

import jax
import jax.numpy as jnp
from jax import lax
from jax.experimental import pallas as pl
from jax.experimental.pallas import tpu as pltpu
from jax.experimental.pallas import tpu_sc as plsc

N_STARS = 1000000
BATCH = 16384
HIST = 200
EPS = 0.001

PAD_ROWS = 7816
PAD_N = PAD_ROWS * 128

N_TOTAL = BATCH * HIST
NW = 32
PER_W = N_TOTAL // NW
CHUNK = 4096
N_CHUNKS = PER_W // CHUNK
LANES = 16


def _combine_body(h_ref, c_ref, o_ref):
    o_ref[...] = jnp.maximum(h_ref[...] + c_ref[...], EPS)


def _combine_tables(hard_p, corr_p):
    return pl.pallas_call(
        _combine_body,
        out_shape=jax.ShapeDtypeStruct((PAD_ROWS, 128), jnp.float32),
    )(hard_p, corr_p)


def _sc_body(freq_hbm, idx_hbm, delta_hbm, out_hbm, idx_v, f_v, d_v, o_v, sem):
    wid = lax.axis_index("s") * 2 + lax.axis_index("c")
    base = wid * PER_W

    def chunk_step(i, carry):
        off = base + i * CHUNK
        pltpu.sync_copy(idx_hbm.at[pl.ds(off, CHUNK)], idx_v)
        gather = pltpu.async_copy(delta_hbm.at[idx_v], d_v, sem)
        pltpu.sync_copy(freq_hbm.at[pl.ds(off, CHUNK)], f_v)
        gather.wait()

        def vec_step(j, c):
            sl = pl.ds(j * LANES, LANES)
            o_v[sl] = jnp.remainder(f_v[sl], d_v[sl])
            return c

        lax.fori_loop(0, CHUNK // LANES, vec_step, 0, unroll=4)
        pltpu.sync_copy(o_v, out_hbm.at[pl.ds(off, CHUNK)])
        return carry

    lax.fori_loop(0, N_CHUNKS, chunk_step, 0)


@functools.partial(
    pl.kernel,
    out_type=jax.ShapeDtypeStruct((N_TOTAL,), jnp.float32),
    mesh=plsc.VectorSubcoreMesh(core_axis_name="c", subcore_axis_name="s"),
    scratch_types=[
        pltpu.VMEM((CHUNK,), jnp.int32),
        pltpu.VMEM((CHUNK,), jnp.float32),
        pltpu.VMEM((CHUNK,), jnp.float32),
        pltpu.VMEM((CHUNK,), jnp.float32),
        pltpu.SemaphoreType.DMA,
    ],
)
def _sc_lookup_rem(freq_hbm, idx_hbm, delta_hbm, out_hbm, idx_v, f_v, d_v, o_v, sem):
    _sc_body(freq_hbm, idx_hbm, delta_hbm, out_hbm, idx_v, f_v, d_v, o_v, sem)


def kernel(frequencies, star_indices, delta_nu_hard, delta_nu_corr):
    hard_p = jnp.pad(delta_nu_hard, (0, PAD_N - N_STARS)).reshape(PAD_ROWS, 128)
    corr_p = jnp.pad(delta_nu_corr, (0, PAD_N - N_STARS)).reshape(PAD_ROWS, 128)
    delta = _combine_tables(hard_p, corr_p).reshape(PAD_N)
    freq_flat = frequencies.reshape(N_TOTAL)
    idx_flat = star_indices.reshape(N_TOTAL).astype(jnp.int32)
    out_flat = _sc_lookup_rem(freq_flat, idx_flat, delta)
    return out_flat.reshape(BATCH, HIST)


# speedup vs baseline: 209.5918x; 1.3937x over previous
import functools

import jax
import jax.numpy as jnp
from jax import lax
from jax.experimental import pallas as pl
from jax.experimental.pallas import tpu as pltpu
from jax.experimental.pallas import tpu_sc as plsc

N_STARS = 1000000
BATCH = 16384
HIST = 200
EPS = 0.001

PAD_ROWS = 7816
PAD_N = PAD_ROWS * 128

N_TOTAL = BATCH * HIST
NW = 32
PER_W = N_TOTAL // NW
CHUNK = 4096
N_CHUNKS = PER_W // CHUNK
LANES = 16


def _combine_body(h_ref, c_ref, o_ref):
    o_ref[...] = jnp.maximum(h_ref[...] + c_ref[...], EPS)


def _combine_tables(hard_p, corr_p):
    return pl.pallas_call(
        _combine_body,
        out_shape=jax.ShapeDtypeStruct((PAD_ROWS, 128), jnp.float32),
    )(hard_p, corr_p)


def _sc_body(freq_hbm, idx_hbm, delta_hbm, out_hbm,
             idx_v, f_v, d_v, o_v, sem_i, sem_f, sem_g, sem_o):
    wid = lax.axis_index("s") * 2 + lax.axis_index("c")
    base = wid * PER_W

    h_in = [None, None]
    h_g = [None, None]
    h_o = [None, None]

    def start_in(i):
        b = i % 2
        off = base + i * CHUNK
        h_in[b] = (
            pltpu.async_copy(idx_hbm.at[pl.ds(off, CHUNK)], idx_v[b], sem_i[b]),
            pltpu.async_copy(freq_hbm.at[pl.ds(off, CHUNK)], f_v[b], sem_f[b]),
        )

    def start_gather(i):
        b = i % 2
        h_in[b][0].wait()
        h_g[b] = pltpu.async_copy(delta_hbm.at[idx_v[b]], d_v[b], sem_g[b])

    def compute(b):
        def vec_step(j, c):
            sl = pl.ds(j * LANES, LANES)
            o_v[b][sl] = jnp.remainder(f_v[b][sl], d_v[b][sl])
            return c

        lax.fori_loop(0, CHUNK // LANES, vec_step, 0, unroll=8)

    start_in(0)
    start_in(1)
    start_gather(0)
    for i in range(N_CHUNKS):
        b = i % 2
        if i + 1 < N_CHUNKS:
            start_gather(i + 1)
        h_g[b].wait()
        h_in[b][1].wait()
        if h_o[b] is not None:
            h_o[b].wait()
        compute(b)
        h_o[b] = pltpu.async_copy(
            o_v[b], out_hbm.at[pl.ds(base + i * CHUNK, CHUNK)], sem_o[b])
        if i + 2 < N_CHUNKS:
            start_in(i + 2)
    h_o[(N_CHUNKS - 1) % 2].wait()
    h_o[N_CHUNKS % 2].wait()


@functools.partial(
    pl.kernel,
    out_type=jax.ShapeDtypeStruct((N_TOTAL,), jnp.float32),
    mesh=plsc.VectorSubcoreMesh(core_axis_name="c", subcore_axis_name="s"),
    scratch_types=[
        [pltpu.VMEM((CHUNK,), jnp.int32)] * 2,
        [pltpu.VMEM((CHUNK,), jnp.float32)] * 2,
        [pltpu.VMEM((CHUNK,), jnp.float32)] * 2,
        [pltpu.VMEM((CHUNK,), jnp.float32)] * 2,
        [pltpu.SemaphoreType.DMA] * 2,
        [pltpu.SemaphoreType.DMA] * 2,
        [pltpu.SemaphoreType.DMA] * 2,
        [pltpu.SemaphoreType.DMA] * 2,
    ],
)
def _sc_lookup_rem(freq_hbm, idx_hbm, delta_hbm, out_hbm,
                   idx_v, f_v, d_v, o_v, sem_i, sem_f, sem_g, sem_o):
    _sc_body(freq_hbm, idx_hbm, delta_hbm, out_hbm,
             idx_v, f_v, d_v, o_v, sem_i, sem_f, sem_g, sem_o)


def kernel(frequencies, star_indices, delta_nu_hard, delta_nu_corr):
    hard_p = jnp.pad(delta_nu_hard, (0, PAD_N - N_STARS)).reshape(PAD_ROWS, 128)
    corr_p = jnp.pad(delta_nu_corr, (0, PAD_N - N_STARS)).reshape(PAD_ROWS, 128)
    delta = _combine_tables(hard_p, corr_p).reshape(PAD_N)
    freq_flat = frequencies.reshape(N_TOTAL)
    idx_flat = star_indices.reshape(N_TOTAL).astype(jnp.int32)
    out_flat = _sc_lookup_rem(freq_flat, idx_flat, delta)
    return out_flat.reshape(BATCH, HIST)


# speedup vs baseline: 233.0396x; 1.1119x over previous
import functools

import jax
import jax.numpy as jnp
from jax import lax
from jax.experimental import pallas as pl
from jax.experimental.pallas import tpu as pltpu
from jax.experimental.pallas import tpu_sc as plsc

N_STARS = 1000000
BATCH = 16384
HIST = 200
EPS = 0.001

PAD_ROWS = 7816
PAD_N = PAD_ROWS * 128

N_TOTAL = BATCH * HIST
NW = 32
PER_W = N_TOTAL // NW
CHUNK = 4096
N_CHUNKS = PER_W // CHUNK
LANES = 16


def _combine_body(h_ref, c_ref, o_ref):
    o_ref[...] = jnp.maximum(h_ref[...] + c_ref[...], EPS)


def _combine_tables(hard_p, corr_p):
    return pl.pallas_call(
        _combine_body,
        out_shape=jax.ShapeDtypeStruct((PAD_ROWS, 128), jnp.float32),
    )(hard_p, corr_p)


def _sc_body(freq_hbm, idx_hbm, delta_hbm, out_hbm,
             idx_v, f_v, d_v, o_v, sem_i, sem_f, sem_g, sem_o):
    wid = lax.axis_index("s") * 2 + lax.axis_index("c")
    base = wid * PER_W

    h_in = [None, None]
    h_g = [None, None]
    h_o = [None, None]

    def start_in(i):
        b = i % 2
        off = base + i * CHUNK
        h_in[b] = (
            pltpu.async_copy(idx_hbm.at[pl.ds(off, CHUNK)], idx_v[b], sem_i[b]),
            pltpu.async_copy(freq_hbm.at[pl.ds(off, CHUNK)], f_v[b], sem_f[b]),
        )

    def start_gather(i):
        b = i % 2
        h_in[b][0].wait()
        h_g[b] = pltpu.async_copy(delta_hbm.at[idx_v[b]], d_v[b], sem_g[b])

    def compute(b):
        @plsc.parallel_loop(0, CHUNK, LANES, unroll=8)
        def _(p):
            sl = pl.ds(p, LANES)
            o_v[b][sl] = lax.rem(f_v[b][sl], d_v[b][sl])

    start_in(0)
    start_in(1)
    start_gather(0)
    for i in range(N_CHUNKS):
        b = i % 2
        if i + 1 < N_CHUNKS:
            start_gather(i + 1)
        h_g[b].wait()
        h_in[b][1].wait()
        if h_o[b] is not None:
            h_o[b].wait()
        compute(b)
        h_o[b] = pltpu.async_copy(
            o_v[b], out_hbm.at[pl.ds(base + i * CHUNK, CHUNK)], sem_o[b])
        if i + 2 < N_CHUNKS:
            start_in(i + 2)
    h_o[(N_CHUNKS - 1) % 2].wait()
    h_o[N_CHUNKS % 2].wait()


@functools.partial(
    pl.kernel,
    out_type=jax.ShapeDtypeStruct((N_TOTAL,), jnp.float32),
    mesh=plsc.VectorSubcoreMesh(core_axis_name="c", subcore_axis_name="s"),
    scratch_types=[
        [pltpu.VMEM((CHUNK,), jnp.int32)] * 2,
        [pltpu.VMEM((CHUNK,), jnp.float32)] * 2,
        [pltpu.VMEM((CHUNK,), jnp.float32)] * 2,
        [pltpu.VMEM((CHUNK,), jnp.float32)] * 2,
        [pltpu.SemaphoreType.DMA] * 2,
        [pltpu.SemaphoreType.DMA] * 2,
        [pltpu.SemaphoreType.DMA] * 2,
        [pltpu.SemaphoreType.DMA] * 2,
    ],
)
def _sc_lookup_rem(freq_hbm, idx_hbm, delta_hbm, out_hbm,
                   idx_v, f_v, d_v, o_v, sem_i, sem_f, sem_g, sem_o):
    _sc_body(freq_hbm, idx_hbm, delta_hbm, out_hbm,
             idx_v, f_v, d_v, o_v, sem_i, sem_f, sem_g, sem_o)


def kernel(frequencies, star_indices, delta_nu_hard, delta_nu_corr):
    hard_p = jnp.pad(delta_nu_hard, (0, PAD_N - N_STARS)).reshape(PAD_ROWS, 128)
    corr_p = jnp.pad(delta_nu_corr, (0, PAD_N - N_STARS)).reshape(PAD_ROWS, 128)
    delta = _combine_tables(hard_p, corr_p).reshape(PAD_N)
    freq_flat = frequencies.reshape(N_TOTAL)
    idx_flat = star_indices.reshape(N_TOTAL).astype(jnp.int32)
    out_flat = _sc_lookup_rem(freq_flat, idx_flat, delta)
    return out_flat.reshape(BATCH, HIST)


# speedup vs baseline: 236.0940x; 1.0131x over previous
import functools

import jax
import jax.numpy as jnp
from jax import lax
from jax.experimental import pallas as pl
from jax.experimental.pallas import tpu as pltpu
from jax.experimental.pallas import tpu_sc as plsc

N_STARS = 1000000
BATCH = 16384
HIST = 200
EPS = 0.001

PAD_ROWS = 7816
PAD_N = PAD_ROWS * 128

N_TOTAL = BATCH * HIST
NW = 32
ROWS_PER_CHUNK = 16
CHUNK = ROWS_PER_CHUNK * HIST
N_CHUNK_ROWS = BATCH // ROWS_PER_CHUNK
N_CHUNKS = N_CHUNK_ROWS // NW
LANES = 16


def _combine_body(h_ref, c_ref, o_ref):
    o_ref[...] = jnp.maximum(h_ref[...] + c_ref[...], EPS)


def _combine_tables(hard_p, corr_p):
    return pl.pallas_call(
        _combine_body,
        out_shape=jax.ShapeDtypeStruct((PAD_ROWS, 128), jnp.float32),
    )(hard_p, corr_p)


def _sc_body(freq_hbm, idx_hbm, delta_hbm, out_hbm,
             idx_v, f_v, d_v, o_v, sem_i, sem_f, sem_g, sem_o):
    wid = lax.axis_index("s") * 2 + lax.axis_index("c")
    base = wid * N_CHUNKS

    def idx_copy(i, b):
        return pltpu.make_async_copy(
            idx_hbm.at[pl.ds((base + i) * CHUNK, CHUNK)], idx_v[b], sem_i[b])

    def f_copy(i, b):
        return pltpu.make_async_copy(
            freq_hbm.at[pl.ds((base + i) * CHUNK, CHUNK)], f_v[b], sem_f[b])

    def g_copy(b):
        return pltpu.make_async_copy(delta_hbm.at[idx_v[b]], d_v[b], sem_g[b])

    def o_copy(i, b):
        return pltpu.make_async_copy(
            o_v[b], out_hbm.at[pl.ds((base + i) * CHUNK, CHUNK)], sem_o[b])

    def compute(b):
        @plsc.parallel_loop(0, CHUNK, LANES, unroll=8)
        def _(p):
            sl = pl.ds(p, LANES)
            o_v[b][sl] = lax.rem(f_v[b][sl], d_v[b][sl])

    idx_copy(0, 0).start()
    f_copy(0, 0).start()
    idx_copy(1, 1).start()
    f_copy(1, 1).start()
    idx_copy(0, 0).wait()
    g_copy(0).start()

    def pair_step(p, carry):
        for b in (0, 1):
            i = 2 * p + b

            @pl.when(i + 1 < N_CHUNKS)
            def _():
                idx_copy(i + 1, 1 - b).wait()
                g_copy(1 - b).start()

            g_copy(b).wait()
            f_copy(i, b).wait()

            @pl.when(i >= 2)
            def _():
                o_copy(i - 2, b).wait()

            compute(b)
            o_copy(i, b).start()

            @pl.when(i + 2 < N_CHUNKS)
            def _():
                idx_copy(i + 2, b).start()
                f_copy(i + 2, b).start()
        return carry

    lax.fori_loop(0, N_CHUNKS // 2, pair_step, 0)
    o_copy(N_CHUNKS - 2, 0).wait()
    o_copy(N_CHUNKS - 1, 1).wait()


@functools.partial(
    pl.kernel,
    out_type=jax.ShapeDtypeStruct((N_TOTAL,), jnp.float32),
    mesh=plsc.VectorSubcoreMesh(core_axis_name="c", subcore_axis_name="s"),
    scratch_types=[
        [pltpu.VMEM((CHUNK,), jnp.int32)] * 2,
        [pltpu.VMEM((CHUNK,), jnp.float32)] * 2,
        [pltpu.VMEM((CHUNK,), jnp.float32)] * 2,
        [pltpu.VMEM((CHUNK,), jnp.float32)] * 2,
        [pltpu.SemaphoreType.DMA] * 2,
        [pltpu.SemaphoreType.DMA] * 2,
        [pltpu.SemaphoreType.DMA] * 2,
        [pltpu.SemaphoreType.DMA] * 2,
    ],
)
def _sc_lookup_rem(freq_hbm, idx_hbm, delta_hbm, out_hbm,
                   idx_v, f_v, d_v, o_v, sem_i, sem_f, sem_g, sem_o):
    _sc_body(freq_hbm, idx_hbm, delta_hbm, out_hbm,
             idx_v, f_v, d_v, o_v, sem_i, sem_f, sem_g, sem_o)


def kernel(frequencies, star_indices, delta_nu_hard, delta_nu_corr):
    hard_p = jnp.pad(delta_nu_hard, (0, PAD_N - N_STARS)).reshape(PAD_ROWS, 128)
    corr_p = jnp.pad(delta_nu_corr, (0, PAD_N - N_STARS)).reshape(PAD_ROWS, 128)
    delta = _combine_tables(hard_p, corr_p).reshape(PAD_N)
    freq_flat = frequencies.reshape(N_TOTAL)
    idx_flat = star_indices.reshape(N_TOTAL).astype(jnp.int32)
    out_flat = _sc_lookup_rem(freq_flat, idx_flat, delta)
    return out_flat.reshape(BATCH, HIST)


# speedup vs baseline: 335.6914x; 1.4219x over previous
import functools

import jax
import jax.numpy as jnp
from jax import lax
from jax.experimental import pallas as pl
from jax.experimental.pallas import tpu as pltpu
from jax.experimental.pallas import tpu_sc as plsc

N_STARS = 1000000
BATCH = 16384
HIST = 200
EPS = 0.001

PAD_ROWS = 7816
PAD_N = PAD_ROWS * 128

N_TOTAL = BATCH * HIST
NW = 32
ROWS_PER_CHUNK = 16
CHUNK = ROWS_PER_CHUNK * HIST
N_CHUNK_ROWS = BATCH // ROWS_PER_CHUNK
N_CHUNKS = N_CHUNK_ROWS // NW
LANES = 16


def _combine_body(h_ref, c_ref, o_ref):
    o_ref[...] = jnp.maximum(h_ref[...] + c_ref[...], EPS)


def _combine_tables(hard_p, corr_p):
    return pl.pallas_call(
        _combine_body,
        out_shape=jax.ShapeDtypeStruct((PAD_ROWS, 128), jnp.float32),
    )(hard_p, corr_p)


def _sc_body(freq_hbm, idx_hbm, delta_hbm, out_hbm,
             idx_v, f_v, d_v, o_v, spmem, stage_v,
             sem_i, sem_f, sem_g, sem_o):
    wid = lax.axis_index("s") * 2 + lax.axis_index("c")
    base = wid * N_CHUNKS

    sid = lax.axis_index("s")
    shard = PAD_N // 16
    piece = shard // 8
    for k in range(8):
        off = sid * shard + k * piece
        pltpu.sync_copy(delta_hbm.at[pl.ds(off, piece)], stage_v)
        pltpu.sync_copy(stage_v, spmem.at[pl.ds(off, piece)])
    plsc.subcore_barrier()

    def idx_copy(i, b):
        return pltpu.make_async_copy(
            idx_hbm.at[pl.ds((base + i) * CHUNK, CHUNK)], idx_v[b], sem_i[b])

    def f_copy(i, b):
        return pltpu.make_async_copy(
            freq_hbm.at[pl.ds((base + i) * CHUNK, CHUNK)], f_v[b], sem_f[b])

    def g_copy(b):
        return pltpu.make_async_copy(spmem.at[idx_v[b]], d_v[b], sem_g[b])

    def o_copy(i, b):
        return pltpu.make_async_copy(
            o_v[b], out_hbm.at[pl.ds((base + i) * CHUNK, CHUNK)], sem_o[b])

    def compute(b):
        @plsc.parallel_loop(0, CHUNK, LANES, unroll=8)
        def _(p):
            sl = pl.ds(p, LANES)
            o_v[b][sl] = lax.rem(f_v[b][sl], d_v[b][sl])

    idx_copy(0, 0).start()
    f_copy(0, 0).start()
    idx_copy(1, 1).start()
    f_copy(1, 1).start()
    idx_copy(0, 0).wait()
    g_copy(0).start()

    def pair_step(p, carry):
        for b in (0, 1):
            i = 2 * p + b

            @pl.when(i + 1 < N_CHUNKS)
            def _():
                idx_copy(i + 1, 1 - b).wait()
                g_copy(1 - b).start()

            g_copy(b).wait()
            f_copy(i, b).wait()

            @pl.when(i >= 2)
            def _():
                o_copy(i - 2, b).wait()

            compute(b)
            o_copy(i, b).start()

            @pl.when(i + 2 < N_CHUNKS)
            def _():
                idx_copy(i + 2, b).start()
                f_copy(i + 2, b).start()
        return carry

    lax.fori_loop(0, N_CHUNKS // 2, pair_step, 0)
    o_copy(N_CHUNKS - 2, 0).wait()
    o_copy(N_CHUNKS - 1, 1).wait()


@functools.partial(
    pl.kernel,
    out_type=jax.ShapeDtypeStruct((N_TOTAL,), jnp.float32),
    mesh=plsc.VectorSubcoreMesh(core_axis_name="c", subcore_axis_name="s"),
    scratch_types=[
        [pltpu.VMEM((CHUNK,), jnp.int32)] * 2,
        [pltpu.VMEM((CHUNK,), jnp.float32)] * 2,
        [pltpu.VMEM((CHUNK,), jnp.float32)] * 2,
        [pltpu.VMEM((CHUNK,), jnp.float32)] * 2,
        pltpu.VMEM_SHARED((PAD_N,), jnp.float32),
        pltpu.VMEM((PAD_N // 128,), jnp.float32),
        [pltpu.SemaphoreType.DMA] * 2,
        [pltpu.SemaphoreType.DMA] * 2,
        [pltpu.SemaphoreType.DMA] * 2,
        [pltpu.SemaphoreType.DMA] * 2,
    ],
)
def _sc_lookup_rem(freq_hbm, idx_hbm, delta_hbm, out_hbm,
                   idx_v, f_v, d_v, o_v, spmem, stage_v,
                   sem_i, sem_f, sem_g, sem_o):
    _sc_body(freq_hbm, idx_hbm, delta_hbm, out_hbm,
             idx_v, f_v, d_v, o_v, spmem, stage_v,
             sem_i, sem_f, sem_g, sem_o)


def kernel(frequencies, star_indices, delta_nu_hard, delta_nu_corr):
    hard_p = jnp.pad(delta_nu_hard, (0, PAD_N - N_STARS)).reshape(PAD_ROWS, 128)
    corr_p = jnp.pad(delta_nu_corr, (0, PAD_N - N_STARS)).reshape(PAD_ROWS, 128)
    delta = _combine_tables(hard_p, corr_p).reshape(PAD_N)
    freq_flat = frequencies.reshape(N_TOTAL)
    idx_flat = star_indices.reshape(N_TOTAL).astype(jnp.int32)
    out_flat = _sc_lookup_rem(freq_flat, idx_flat, delta)
    return out_flat.reshape(BATCH, HIST)


# speedup vs baseline: 347.3266x; 1.0347x over previous
import functools

import jax
import jax.numpy as jnp
from jax import lax
from jax.experimental import pallas as pl
from jax.experimental.pallas import tpu as pltpu
from jax.experimental.pallas import tpu_sc as plsc

N_STARS = 1000000
BATCH = 16384
HIST = 200
EPS = 0.001

PAD_ROWS = 7816
PAD_N = PAD_ROWS * 128

N_TOTAL = BATCH * HIST
NW = 32
LANES = 16
VIEW_ROWS = N_TOTAL // 128
CROWS = 40
CHUNK = CROWS * 128
N_CHUNKS = N_TOTAL // CHUNK // NW


def _combine_body(h_ref, c_ref, o_ref):
    o_ref[...] = jnp.maximum(h_ref[...] + c_ref[...], EPS)


def _combine_tables(hard_p, corr_p):
    return pl.pallas_call(
        _combine_body,
        out_shape=jax.ShapeDtypeStruct((PAD_ROWS, 128), jnp.float32),
    )(hard_p, corr_p)


def _sc_body(freq_hbm, idx_hbm, delta_hbm, out_hbm,
             idx_v, f_v, d_v, o_v, spmem, stage_v,
             sem_i, sem_f, sem_g, sem_o):
    wid = lax.axis_index("s") * 2 + lax.axis_index("c")
    base = wid * N_CHUNKS

    sid = lax.axis_index("s")
    shard = PAD_N // 16
    piece = shard // 8
    for k in range(8):
        off = sid * shard + k * piece
        pltpu.sync_copy(delta_hbm.at[pl.ds(off, piece)], stage_v)
        pltpu.sync_copy(stage_v, spmem.at[pl.ds(off, piece)])
    plsc.subcore_barrier()

    def idx_copy(i, b):
        return pltpu.make_async_copy(
            idx_hbm.at[pl.ds((base + i) * CHUNK, CHUNK)], idx_v[b], sem_i[b])

    def f_copy(i, b):
        return pltpu.make_async_copy(
            freq_hbm.at[pl.ds((base + i) * CROWS, CROWS)], f_v[b], sem_f[b])

    def g_copy(b):
        return pltpu.make_async_copy(spmem.at[idx_v[b]], d_v[b], sem_g[b])

    def o_copy(i, b):
        return pltpu.make_async_copy(
            o_v[b], out_hbm.at[pl.ds((base + i) * CROWS, CROWS)], sem_o[b])

    def compute(b):
        @plsc.parallel_loop(0, CROWS, 1)
        def _(r):
            for j in range(8):
                c = j * LANES
                o_v[b][r, pl.ds(c, LANES)] = lax.rem(
                    f_v[b][r, pl.ds(c, LANES)],
                    d_v[b][pl.ds(r * 128 + c, LANES)])

    idx_copy(0, 0).start()
    f_copy(0, 0).start()
    idx_copy(1, 1).start()
    f_copy(1, 1).start()
    idx_copy(0, 0).wait()
    g_copy(0).start()

    def pair_step(p, carry):
        for b in (0, 1):
            i = 2 * p + b

            @pl.when(i + 1 < N_CHUNKS)
            def _():
                idx_copy(i + 1, 1 - b).wait()
                g_copy(1 - b).start()

            g_copy(b).wait()
            f_copy(i, b).wait()

            @pl.when(i >= 2)
            def _():
                o_copy(i - 2, b).wait()

            compute(b)
            o_copy(i, b).start()

            @pl.when(i + 2 < N_CHUNKS)
            def _():
                idx_copy(i + 2, b).start()
                f_copy(i + 2, b).start()
        return carry

    lax.fori_loop(0, N_CHUNKS // 2, pair_step, 0)
    o_copy(N_CHUNKS - 2, 0).wait()
    o_copy(N_CHUNKS - 1, 1).wait()


@functools.partial(
    pl.kernel,
    out_type=jax.ShapeDtypeStruct((VIEW_ROWS, 128), jnp.float32),
    mesh=plsc.VectorSubcoreMesh(core_axis_name="c", subcore_axis_name="s"),
    scratch_types=[
        [pltpu.VMEM((CHUNK,), jnp.int32)] * 2,
        [pltpu.VMEM((CROWS, 128), jnp.float32)] * 2,
        [pltpu.VMEM((CHUNK,), jnp.float32)] * 2,
        [pltpu.VMEM((CROWS, 128), jnp.float32)] * 2,
        pltpu.VMEM_SHARED((PAD_N,), jnp.float32),
        pltpu.VMEM((PAD_N // 128,), jnp.float32),
        [pltpu.SemaphoreType.DMA] * 2,
        [pltpu.SemaphoreType.DMA] * 2,
        [pltpu.SemaphoreType.DMA] * 2,
        [pltpu.SemaphoreType.DMA] * 2,
    ],
)
def _sc_lookup_rem(freq_hbm, idx_hbm, delta_hbm, out_hbm,
                   idx_v, f_v, d_v, o_v, spmem, stage_v,
                   sem_i, sem_f, sem_g, sem_o):
    _sc_body(freq_hbm, idx_hbm, delta_hbm, out_hbm,
             idx_v, f_v, d_v, o_v, spmem, stage_v,
             sem_i, sem_f, sem_g, sem_o)


def kernel(frequencies, star_indices, delta_nu_hard, delta_nu_corr):
    hard_p = jnp.pad(delta_nu_hard, (0, PAD_N - N_STARS)).reshape(PAD_ROWS, 128)
    corr_p = jnp.pad(delta_nu_corr, (0, PAD_N - N_STARS)).reshape(PAD_ROWS, 128)
    delta = _combine_tables(hard_p, corr_p).reshape(PAD_N)
    freq_v = frequencies.reshape(VIEW_ROWS, 128)
    idx_flat = star_indices.reshape(N_TOTAL).astype(jnp.int32)
    out_v = _sc_lookup_rem(freq_v, idx_flat, delta)
    return out_v.reshape(BATCH, HIST)
